# baseline (speedup 1.0000x reference)
import functools

import jax
import jax.numpy as jnp
from jax import lax
from jax.experimental import pallas as pl
from jax.experimental.pallas import tpu as pltpu
from jax.experimental.pallas import tpu_sc as plsc

F32 = jnp.float32
ROW_TILE = 256


def _ceil_to(a, m):
    return (a + m - 1) // m * m


def _pad2(x, rows, cols):
    pr, pc = rows - x.shape[0], cols - x.shape[1]
    if pr == 0 and pc == 0:
        return x
    return jnp.pad(x, ((0, pr), (0, pc)))


def _silu(v):
    return v / (1.0 + jnp.exp(-v))


def _finalize_stats(s, q, n, eps=1e-5):
    mu = (s / n).reshape(-1)
    var = (q / n).reshape(-1) - mu * mu
    rstd = 1.0 / jnp.sqrt(jnp.maximum(var, 0.0) + eps)
    return mu, rstd


def _mm(x, w, b, *, stats=False, norm=None):
    R, din = x.shape
    dout = w.shape[1]
    rt = min(ROW_TILE, _ceil_to(R, 8))
    rp = _ceil_to(R, rt)
    dinp = _ceil_to(din, 8)
    xp = _pad2(x, rp, dinp)
    wp = _pad2(w, dinp, dout)
    b2 = b.reshape(1, dout)
    grid = (rp // rt,)

    args = [xp]
    in_specs = [pl.BlockSpec((rt, dinp), lambda i: (i, 0))]
    if norm is not None:
        for v in norm:
            args.append(_pad2(v.reshape(1, din), 1, dinp))
            in_specs.append(pl.BlockSpec((1, dinp), lambda i: (0, 0)))
    args += [wp, b2]
    in_specs += [pl.BlockSpec((dinp, dout), lambda i: (0, 0)),
                 pl.BlockSpec((1, dout), lambda i: (0, 0))]

    out_shape = [jax.ShapeDtypeStruct((rp, dout), F32)]
    out_specs = [pl.BlockSpec((rt, dout), lambda i: (i, 0))]
    if stats:
        out_shape += [jax.ShapeDtypeStruct((1, dout), F32)] * 2
        out_specs += [pl.BlockSpec((1, dout), lambda i: (0, 0))] * 2

    def body(*refs):
        if norm is not None:
            x_ref, mu_ref, rstd_ref, g_ref, be_ref, w_ref, b_ref = refs[:7]
            rest = refs[7:]
        else:
            x_ref, w_ref, b_ref = refs[:3]
            rest = refs[3:]
        xv = x_ref[...]
        if norm is not None:
            xv = (xv - mu_ref[...]) * rstd_ref[...] * g_ref[...] + be_ref[...]
            xv = _silu(xv)
        t = jnp.dot(xv, w_ref[...], preferred_element_type=F32) + b_ref[...]
        if stats:
            out_ref, s_ref, q_ref = rest
        else:
            (out_ref,) = rest
        out_ref[...] = t
        if stats:
            i = pl.program_id(0)
            rows = i * rt + lax.broadcasted_iota(jnp.int32, (rt, dout), 0)
            tm = jnp.where(rows < R, t, 0.0)

            @pl.when(i == 0)
            def _():
                s_ref[...] = jnp.zeros_like(s_ref)
                q_ref[...] = jnp.zeros_like(q_ref)

            s_ref[...] += jnp.sum(tm, axis=0, keepdims=True)
            q_ref[...] += jnp.sum(tm * tm, axis=0, keepdims=True)

    outs = pl.pallas_call(body, grid=grid, in_specs=in_specs,
                          out_specs=out_specs, out_shape=out_shape)(*args)
    if stats:
        return outs[0][:R], outs[1], outs[2]
    return outs[0][:R]


def _mm4(x, ws, bs):
    R, d = x.shape
    rt = min(ROW_TILE, _ceil_to(R, 8))
    rp = _ceil_to(R, rt)
    xp = _pad2(x, rp, d)
    grid = (rp // rt,)

    def body(x_ref, w_ref, b_ref, o0, o1, o2, o3):
        xv = x_ref[...]
        outs = (o0, o1, o2, o3)
        for k in range(4):
            outs[k][...] = (jnp.dot(xv, w_ref[k], preferred_element_type=F32)
                            + b_ref[k])

    outs = pl.pallas_call(
        body, grid=grid,
        in_specs=[pl.BlockSpec((rt, d), lambda i: (i, 0)),
                  pl.BlockSpec((4, d, d), lambda i: (0, 0, 0)),
                  pl.BlockSpec((4, 1, d), lambda i: (0, 0, 0))],
        out_specs=[pl.BlockSpec((rt, d), lambda i: (i, 0))] * 4,
        out_shape=[jax.ShapeDtypeStruct((rp, d), F32)] * 4,
    )(xp, ws, bs)
    return tuple(o[:R] for o in outs)


def _combine(xup, a, s):
    R, d = xup.shape
    rt = min(ROW_TILE, _ceil_to(R, 8))
    rp = _ceil_to(R, rt)
    grid = (rp // rt,)

    def body(u_ref, a_ref, s_ref, o_ref, cs_ref, cq_ref):
        t = u_ref[...] + a_ref[...] / (s_ref[...] + 1e-6)
        o_ref[...] = t
        i = pl.program_id(0)
        rows = i * rt + lax.broadcasted_iota(jnp.int32, (rt, d), 0)
        tm = jnp.where(rows < R, t, 0.0)

        @pl.when(i == 0)
        def _():
            cs_ref[...] = jnp.zeros_like(cs_ref)
            cq_ref[...] = jnp.zeros_like(cq_ref)

        cs_ref[...] += jnp.sum(tm, axis=0, keepdims=True)
        cq_ref[...] += jnp.sum(tm * tm, axis=0, keepdims=True)

    outs = pl.pallas_call(
        body, grid=grid,
        in_specs=[pl.BlockSpec((rt, d), lambda i: (i, 0))] * 3,
        out_specs=[pl.BlockSpec((rt, d), lambda i: (i, 0)),
                   pl.BlockSpec((1, d), lambda i: (0, 0)),
                   pl.BlockSpec((1, d), lambda i: (0, 0))],
        out_shape=[jax.ShapeDtypeStruct((rp, d), F32),
                   jax.ShapeDtypeStruct((1, d), F32),
                   jax.ShapeDtypeStruct((1, d), F32)],
    )(_pad2(xup, rp, d), _pad2(a, rp, d), _pad2(s, rp, d))
    return outs[0][:R], outs[1], outs[2]


def _norm_apply(t, mu, rstd, g, be, residual=None, colsum=False):
    R, d = t.shape
    rt = min(ROW_TILE, _ceil_to(R, 8))
    rp = _ceil_to(R, rt)
    grid = (rp // rt,)
    args = [_pad2(t, rp, d)]
    in_specs = [pl.BlockSpec((rt, d), lambda i: (i, 0))]
    for v in (mu, rstd, g, be):
        args.append(v.reshape(1, d))
        in_specs.append(pl.BlockSpec((1, d), lambda i: (0, 0)))
    if residual is not None:
        args.append(_pad2(residual, rp, d))
        in_specs.append(pl.BlockSpec((rt, d), lambda i: (i, 0)))

    out_shape = [jax.ShapeDtypeStruct((rp, d), F32)]
    out_specs = [pl.BlockSpec((rt, d), lambda i: (i, 0))]
    if colsum:
        out_shape.append(jax.ShapeDtypeStruct((1, d), F32))
        out_specs.append(pl.BlockSpec((1, d), lambda i: (0, 0)))

    def body(*refs):
        t_ref, mu_ref, rstd_ref, g_ref, be_ref = refs[:5]
        rest = list(refs[5:])
        v = (t_ref[...] - mu_ref[...]) * rstd_ref[...] * g_ref[...] + be_ref[...]
        v = _silu(v)
        if residual is not None:
            v = rest.pop(0)[...] + v
        o_ref = rest.pop(0)
        o_ref[...] = v
        if colsum:
            cs_ref = rest.pop(0)
            i = pl.program_id(0)
            rows = i * rt + lax.broadcasted_iota(jnp.int32, (rt, d), 0)
            vm = jnp.where(rows < R, v, 0.0)

            @pl.when(i == 0)
            def _():
                cs_ref[...] = jnp.zeros_like(cs_ref)

            cs_ref[...] += jnp.sum(vm, axis=0, keepdims=True)

    outs = pl.pallas_call(body, grid=grid, in_specs=in_specs,
                          out_specs=out_specs, out_shape=out_shape)(*args)
    if colsum:
        return outs[0][:R], outs[1]
    return outs[0][:R]


def _messages(es, ed, bh, yg, src, dst, n_out):
    m = es[src] + ed[dst] + yg
    sigma = jax.nn.sigmoid(m)
    a = jax.ops.segment_sum(sigma * bh[src], dst, num_segments=n_out)
    s = jax.ops.segment_sum(sigma, dst, num_segments=n_out)
    ms = jnp.sum(m, axis=0, keepdims=True)
    mq = jnp.sum(m * m, axis=0, keepdims=True)
    return m, a, s, ms, mq


def _egc(p, x, y, src, dst, n_nodes):
    e_rows = y.shape[0]
    ws = jnp.stack([p["src_gate"]["W"], p["dst_gate"]["W"],
                    p["src_update"]["W"], p["dst_update"]["W"]])
    bs = jnp.stack([p["src_gate"]["b"].reshape(1, -1),
                    p["dst_gate"]["b"].reshape(1, -1),
                    p["src_update"]["b"].reshape(1, -1),
                    p["dst_update"]["b"].reshape(1, -1)])
    es, ed, bh, xup = _mm4(x, ws, bs)
    yg = _mm(y, p["edge_gate"]["W"], p["edge_gate"]["b"])
    m, a, s, ms, mq = _messages(es, ed, bh, yg, src, dst, n_nodes)
    xn, cs, cq = _combine(xup, a, s)
    mu, rstd = _finalize_stats(cs, cq, n_nodes)
    x_out = _norm_apply(xn, mu, rstd, p["bn_nodes"]["g"], p["bn_nodes"]["be"],
                        residual=x)
    mmu, mrstd = _finalize_stats(ms, mq, e_rows)
    y_out = _norm_apply(m, mmu, mrstd, p["bn_edges"]["g"], p["bn_edges"]["be"],
                        residual=y)
    return x_out, y_out


def _rbf(x, vmin, vmax, bins):
    centers = jnp.linspace(vmin, vmax, bins)
    gamma = 1.0 / (centers[1] - centers[0])
    return jnp.exp(-gamma * (x[:, None] - centers) ** 2)


def _mlp_pair(x, p1, p2, n):
    t1, s1, q1 = _mm(x, p1["lin"]["W"], p1["lin"]["b"], stats=True)
    mu1, rstd1 = _finalize_stats(s1, q1, n)
    t2, s2, q2 = _mm(t1, p2["lin"]["W"], p2["lin"]["b"], stats=True,
                     norm=(mu1, rstd1, p1["bn"]["g"], p1["bn"]["be"]))
    mu2, rstd2 = _finalize_stats(s2, q2, n)
    return _norm_apply(t2, mu2, rstd2, p2["bn"]["g"], p2["bn"]["be"])


def kernel(atom_features, r, angle_h, edge_index, lg_edge_index, params):
    n = atom_features.shape[0]
    e = r.shape[0]
    t_ang = angle_h.shape[0]

    pa = params["atom_emb"]
    t0, s0, q0 = _mm(atom_features, pa["lin"]["W"], pa["lin"]["b"], stats=True)
    mu0, rstd0 = _finalize_stats(s0, q0, n)
    x = _norm_apply(t0, mu0, rstd0, pa["bn"]["g"], pa["bn"]["be"])

    bondlength = jnp.sqrt(jnp.sum(r * r, axis=1))
    y = _mlp_pair(_rbf(bondlength, 0.0, 8.0, 80),
                  params["edge_mlp1"], params["edge_mlp2"], e)
    z = _mlp_pair(_rbf(angle_h, -1.0, 1.0, 40),
                  params["ang_mlp1"], params["ang_mlp2"], t_ang)

    src, dst = edge_index[0], edge_index[1]
    lsrc, ldst = lg_edge_index[0], lg_edge_index[1]
    for lp in params["layers"]:
        x, m = _egc(lp["node"], x, y, src, dst, n)
        y, z = _egc(lp["edge"], m, z, lsrc, ldst, e)

    hpool = jnp.mean(x, axis=0)
    return jnp.squeeze(hpool @ params["fc"]["W"] + params["fc"]["b"])
